# Initial kernel scaffold; baseline (speedup 1.0000x reference)
import jax
import jax.numpy as jnp
from jax.experimental import pallas as pl


def kernel(x, mem, w1, b1, w2, b2, w3, b3, d1, bd1, d2, bd2, d3, bd3):
    raise NotImplementedError("write your pallas kernel here")


# speedup vs baseline: 1.0468x; 1.0468x over previous
import functools

import jax
import jax.numpy as jnp
from jax.experimental import pallas as pl
from jax.experimental.pallas import tpu as pltpu

_N, _C, _H, _W = 2000, 512, 7, 7
_B = 16
_D = _C * _H * _W


def _conv(x, w, b, stride):
    y = jax.lax.conv_general_dilated(x, w, (stride, stride), 'SAME',
                                     dimension_numbers=('NCHW', 'OIHW', 'NCHW'))
    return y + b[None, :, None, None]


def _resize(x, factor):
    b, c, h, w = x.shape
    return jax.image.resize(x, (b, c, h * factor, w * factor), method='nearest')


def _read_body(nblk, z_ref, m_ref, o_ref, mx_ref, l_ref, acc_ref):
    j = pl.program_id(0)

    @pl.when(j == 0)
    def _init():
        mx_ref[...] = jnp.full_like(mx_ref, -jnp.inf)
        l_ref[...] = jnp.zeros_like(l_ref)
        acc_ref[...] = jnp.zeros_like(acc_ref)

    blk = m_ref[...]
    z = z_ref[...]
    dot = jax.lax.dot_general(z, blk, (((1,), (1,)), ((), ())),
                              preferred_element_type=jnp.float32)
    mnorm = jnp.sqrt(jnp.sum(blk * blk, axis=1)) + 1e-8
    znorm = jnp.sqrt(jnp.sum(z * z, axis=1, keepdims=True)) + 1e-8
    s = (dot / (znorm * mnorm[None, :])) * 10.0

    m_prev = mx_ref[...]
    m_new = jnp.maximum(m_prev, jnp.max(s, axis=1, keepdims=True))
    alpha = jnp.exp(m_prev - m_new)
    p = jnp.exp(s - m_new)
    l_ref[...] = l_ref[...] * alpha + jnp.sum(p, axis=1, keepdims=True)
    acc_ref[...] = acc_ref[...] * alpha + jax.lax.dot_general(
        p, blk, (((1,), (0,)), ((), ())), preferred_element_type=jnp.float32)
    mx_ref[...] = m_new

    @pl.when(j == nblk - 1)
    def _fin():
        o_ref[...] = acc_ref[...] / l_ref[...]


def _attention_read(z, mflat):
    n, d = mflat.shape
    bq = z.shape[0]
    k = 200
    nblk = n // k
    return pl.pallas_call(
        functools.partial(_read_body, nblk),
        grid=(nblk,),
        in_specs=[pl.BlockSpec((bq, d), lambda j: (0, 0)),
                  pl.BlockSpec((k, d), lambda j: (j, 0))],
        out_specs=pl.BlockSpec((bq, d), lambda j: (0, 0)),
        out_shape=jax.ShapeDtypeStruct((bq, d), jnp.float32),
        scratch_shapes=[pltpu.VMEM((bq, 1), jnp.float32),
                        pltpu.VMEM((bq, 1), jnp.float32),
                        pltpu.VMEM((bq, d), jnp.float32)],
    )(z, mflat)


def kernel(x, mem, w1, b1, w2, b2, w3, b3, d1, bd1, d2, bd2, d3, bd3):
    h1 = jax.nn.relu(_conv(x, w1, b1, 4))
    h2 = jax.nn.relu(_conv(h1, w2, b2, 4))
    coding = jax.nn.relu(_conv(h2, w3, b3, 2))
    z = coding.reshape(coding.shape[0], -1)
    mflat = mem.reshape(mem.shape[0], -1)
    read = _attention_read(z, mflat)
    states = read.reshape(coding.shape)
    g1 = jax.nn.relu(_conv(_resize(states, 2), d1, bd1, 1))
    g2 = jax.nn.relu(_conv(_resize(g1, 4), d2, bd2, 1))
    output = _conv(_resize(g2, 4), d3, bd3, 1)
    return output
